# baseline (speedup 1.0000x reference)
import functools

import jax
import jax.numpy as jnp
from jax import lax
from jax.experimental import pallas as pl
from jax.experimental.pallas import tpu as pltpu
from jax.experimental.pallas import tpu_sc as plsc

B, T, N = 16, 512, 512
FS = 32
NS = N // FS
SP = NS + 1
PAD = -1.0

_ROWS = T // 2


def _sc_patch_body(spikes, pad, out, pad_v, cur_v, prv_v, sem):
    wid = lax.axis_index("s") * 2 + lax.axis_index("c")
    b = wid // 2
    t0 = pl.multiple_of((wid % 2) * _ROWS, _ROWS)

    big = pltpu.async_copy(
        spikes.at[b, pl.ds(t0, _ROWS)], out.at[b, pl.ds(t0, _ROWS)], sem)

    pltpu.sync_copy(pad, pad_v.at[pl.ds(0, 16)])
    lanes = lax.iota(jnp.int32, 16)
    p = pad_v[pl.ds(b, 16)][0]
    psl = p % FS
    sidx = p // FS
    do_fix = (psl != 0) & (sidx < NS)

    @pl.when(do_fix)
    def _fix():
        cc = sidx // 4
        g4 = sidx % 4
        pc = jnp.maximum(sidx - 1, 0) // 4
        c_cur = pl.multiple_of(cc * 128, 128)
        c_prv = pl.multiple_of(pc * 128, 128)
        pltpu.sync_copy(
            spikes.at[b, pl.ds(t0, _ROWS), pl.ds(c_cur, 128)], cur_v)
        pltpu.sync_copy(
            spikes.at[b, pl.ds(t0, _ROWS), pl.ds(c_prv, 128)], prv_v)

        psl_s = jnp.full((16,), psl)
        mf = jnp.full((16,), (sidx == 0).astype(jnp.float32))

        for g in range(4):
            @pl.when(g4 == g)
            def _grp():
                def row(r, carry):
                    for h in range(2):
                        j = lanes + (16 * h)
                        cur = cur_v[r, pl.ds(g * FS + 16 * h, 16)]
                        if g > 0:
                            prv = cur_v[r, pl.ds((g - 1) * FS + 16 * h, 16)]
                        else:
                            prv = prv_v[r, pl.ds(3 * FS + 16 * h, 16)]
                        prv = prv * (1.0 - mf) + PAD * mf
                        cur_v[r, pl.ds(g * FS + 16 * h, 16)] = (
                            jnp.where(j < psl_s, cur, prv))
                    return carry

                lax.fori_loop(0, _ROWS, row, 0)

        big.wait()
        pltpu.sync_copy(
            cur_v, out.at[b, pl.ds(t0, _ROWS), pl.ds(c_cur, 128)])

    @pl.when(jnp.logical_not(do_fix))
    def _nofix():
        big.wait()


@functools.partial(
    pl.kernel,
    out_type=jax.ShapeDtypeStruct((B, T, N), jnp.float32),
    mesh=plsc.VectorSubcoreMesh(core_axis_name="c", subcore_axis_name="s"),
    scratch_types=[
        pltpu.VMEM((32,), jnp.int32),
        pltpu.VMEM((_ROWS, 128), jnp.float32),
        pltpu.VMEM((_ROWS, 128), jnp.float32),
        pltpu.SemaphoreType.DMA,
    ],
)
def _sc_patchify(spikes, pad, out, pad_v, cur_v, prv_v, sem):
    _sc_patch_body(spikes, pad, out, pad_v, cur_v, prv_v, sem)


def _tc_body(tm_ref, sm_ref, smask_ref, tmask_ref, ss_ref, ts_ref):
    li = lax.broadcasted_iota(jnp.int32, (T, SP), 1)

    tm = tm_ref[0]
    tmask_ref[0] = jnp.where(li == 0, 1, jnp.broadcast_to(tm, (T, SP)))

    sm = sm_ref[0]
    s_col = jnp.max(sm, axis=1, keepdims=True).astype(jnp.float32)
    s_any = lax.dot_general(
        s_col, jnp.eye(NS, dtype=jnp.float32),
        (((0,), (0,)), ((), ())),
        preferred_element_type=jnp.float32).astype(jnp.int32)
    s_row = jnp.concatenate(
        [jnp.ones((1, 1), jnp.int32), s_any], axis=1)
    smask_ref[0] = jnp.broadcast_to(s_row, (T, SP))

    ss_ref[0] = li
    ts_ref[0] = lax.broadcasted_iota(jnp.int32, (T, SP), 0)


def _tc_masks(time_attn_mask, space_attn_mask):
    tm3 = time_attn_mask.reshape(B, T, 1)
    sm3 = space_attn_mask.reshape(B, NS, FS)
    return pl.pallas_call(
        _tc_body,
        grid=(B,),
        in_specs=[
            pl.BlockSpec((1, T, 1), lambda b: (b, 0, 0)),
            pl.BlockSpec((1, NS, FS), lambda b: (b, 0, 0)),
        ],
        out_specs=[
            pl.BlockSpec((1, T, SP), lambda b: (b, 0, 0)),
            pl.BlockSpec((1, T, SP), lambda b: (b, 0, 0)),
            pl.BlockSpec((1, T, SP), lambda b: (b, 0, 0)),
            pl.BlockSpec((1, T, SP), lambda b: (b, 0, 0)),
        ],
        out_shape=[
            jax.ShapeDtypeStruct((B, T, SP), jnp.int32),
            jax.ShapeDtypeStruct((B, T, SP), jnp.int32),
            jax.ShapeDtypeStruct((B, T, SP), jnp.int32),
            jax.ShapeDtypeStruct((B, T, SP), jnp.int32),
        ],
        compiler_params=pltpu.CompilerParams(
            dimension_semantics=("arbitrary",),
        ),
    )(tm3, sm3)


def kernel(spikes, pad_space_len, pad_time_len, time_attn_mask,
           space_attn_mask):
    del pad_time_len
    patches = _sc_patchify(spikes, pad_space_len)
    smask, tmask, ss, ts = _tc_masks(time_attn_mask, space_attn_mask)
    return (patches.reshape(B, T * NS, FS),
            smask.reshape(B, T * SP),
            tmask.reshape(B, T * SP),
            ss.reshape(B, T * SP),
            ts.reshape(B, T * SP))


# speedup vs baseline: 5.5383x; 5.5383x over previous
import jax
import jax.numpy as jnp
from jax import lax
from jax.experimental import pallas as pl
from jax.experimental.pallas import tpu as pltpu

B, T, N = 16, 512, 512
FS = 32
NS = N // FS
SP = NS + 1
PAD = -1.0


def _body(psl_ref, spikes_ref, tm_ref, sm_ref,
          patches_ref, smask_ref, tmask_ref, ss_ref, ts_ref):
    b = pl.program_id(0)
    p = psl_ref[b]
    psl = p % FS
    sidx = p // FS
    do_fix = (psl != 0) & (sidx < NS)

    x = spikes_ref[0]
    lane = lax.broadcasted_iota(jnp.int32, (T, N), 1)
    g = lane // FS
    j = lane - g * FS
    prev = jnp.concatenate(
        [jnp.full((T, FS), PAD, jnp.float32), x[:, : N - FS]], axis=1)
    blended = jnp.where(j < psl, x, prev)
    fixmask = (g == sidx) & do_fix
    patches_ref[0] = jnp.where(fixmask, blended, x).reshape(T * 4, N // 4)

    li = lax.broadcasted_iota(jnp.int32, (T, SP), 1)

    tm = tm_ref[0]
    tmask_ref[0] = jnp.where(li == 0, 1, jnp.broadcast_to(tm, (T, SP)))

    sm = sm_ref[0]
    s_col = jnp.max(sm, axis=1, keepdims=True).astype(jnp.float32)
    s_any = lax.dot_general(
        s_col, jnp.eye(NS, dtype=jnp.float32),
        (((0,), (0,)), ((), ())),
        preferred_element_type=jnp.float32).astype(jnp.int32)
    s_row = jnp.concatenate(
        [jnp.ones((1, 1), jnp.int32), s_any], axis=1)
    smask_ref[0] = jnp.broadcast_to(s_row, (T, SP))

    ss_ref[0] = li
    ts_ref[0] = lax.broadcasted_iota(jnp.int32, (T, SP), 0)


def kernel(spikes, pad_space_len, pad_time_len, time_attn_mask,
           space_attn_mask):
    del pad_time_len
    tm3 = time_attn_mask.reshape(B, T, 1)
    sm3 = space_attn_mask.reshape(B, NS, FS)

    grid_spec = pltpu.PrefetchScalarGridSpec(
        num_scalar_prefetch=1,
        grid=(B,),
        in_specs=[
            pl.BlockSpec((1, T, N), lambda b, psl: (b, 0, 0)),
            pl.BlockSpec((1, T, 1), lambda b, psl: (b, 0, 0)),
            pl.BlockSpec((1, NS, FS), lambda b, psl: (b, 0, 0)),
        ],
        out_specs=[
            pl.BlockSpec((1, T * 4, N // 4), lambda b, psl: (b, 0, 0)),
            pl.BlockSpec((1, T, SP), lambda b, psl: (b, 0, 0)),
            pl.BlockSpec((1, T, SP), lambda b, psl: (b, 0, 0)),
            pl.BlockSpec((1, T, SP), lambda b, psl: (b, 0, 0)),
            pl.BlockSpec((1, T, SP), lambda b, psl: (b, 0, 0)),
        ],
    )
    patches, smask, tmask, ss, ts = pl.pallas_call(
        _body,
        grid_spec=grid_spec,
        out_shape=[
            jax.ShapeDtypeStruct((B, T * 4, N // 4), jnp.float32),
            jax.ShapeDtypeStruct((B, T, SP), jnp.int32),
            jax.ShapeDtypeStruct((B, T, SP), jnp.int32),
            jax.ShapeDtypeStruct((B, T, SP), jnp.int32),
            jax.ShapeDtypeStruct((B, T, SP), jnp.int32),
        ],
        compiler_params=pltpu.CompilerParams(
            dimension_semantics=("arbitrary",),
        ),
    )(pad_space_len, spikes, tm3, sm3)

    return (patches.reshape(B, T * NS, FS),
            smask.reshape(B, T * SP),
            tmask.reshape(B, T * SP),
            ss.reshape(B, T * SP),
            ts.reshape(B, T * SP))
